# baseline (speedup 1.0000x reference)
import functools

import jax
import jax.numpy as jnp
from jax import lax
from jax.experimental import pallas as pl
from jax.experimental.pallas import tpu as pltpu
from jax.experimental.pallas import tpu_sc as plsc

B = 128
V = 100000
TOP_K = 8
RB = 8
GSTEPS = B // RB


def _gather_kernel(pred_hbm, ytrue_hbm, out_hbm, yt_v, idx_v, tval_v, sem):
    wid = lax.axis_index("s") * 2 + lax.axis_index("c")

    @pl.when(wid == 0)
    def _():
        pltpu.sync_copy(ytrue_hbm, yt_v)
        lanes = lax.iota(jnp.int32, 16)
        for g in range(B // 16):
            rows = lanes + g * 16
            idx_v[pl.ds(g * 16, 16)] = rows * V + yt_v[pl.ds(g * 16, 16)]
        pltpu.async_copy(pred_hbm.at[idx_v], tval_v, sem).wait()
        pltpu.sync_copy(tval_v, out_hbm)


@jax.jit
def _gather_targets(pred_flat, ytrue):
    mesh = plsc.VectorSubcoreMesh(core_axis_name="c", subcore_axis_name="s")
    kern = functools.partial(
        pl.kernel,
        mesh=mesh,
        compiler_params=pltpu.CompilerParams(needs_layout_passes=False),
        out_type=jax.ShapeDtypeStruct((B,), jnp.float32),
        scratch_types=[
            pltpu.VMEM((B,), jnp.int32),
            pltpu.VMEM((B,), jnp.int32),
            pltpu.VMEM((B,), jnp.float32),
            pltpu.SemaphoreType.DMA,
        ],
    )(_gather_kernel)
    return kern(pred_flat, ytrue)


def _count_kernel(pred_ref, t_ref, yt_ref, out_ref, s_ref):
    g = pl.program_id(0)

    @pl.when(g == 0)
    def _():
        s_ref[0] = 0

    v = pred_ref[...]
    col = lax.broadcasted_iota(jnp.int32, (RB, V), 1).astype(jnp.float32)
    t = t_ref[:, 0:1]
    yt = yt_ref[:, 0:1]
    m = (v > t) | ((v == t) & (col < yt))
    rank = jnp.sum(m.astype(jnp.int32), axis=1, keepdims=True)
    s_ref[0] += jnp.sum((rank < TOP_K).astype(jnp.int32))

    @pl.when(g == GSTEPS - 1)
    def _():
        out_ref[...] = jnp.full(
            (1, 1), s_ref[0].astype(jnp.float32) / jnp.float32(B))


@jax.jit
def _count_ranks(y_pred, tvals, ytrue_f):
    return pl.pallas_call(
        _count_kernel,
        grid=(GSTEPS,),
        in_specs=[
            pl.BlockSpec((RB, V), lambda g: (g, 0)),
            pl.BlockSpec((RB, 128), lambda g: (g, 0)),
            pl.BlockSpec((RB, 128), lambda g: (g, 0)),
        ],
        out_specs=pl.BlockSpec((1, 1), lambda g: (0, 0)),
        out_shape=jax.ShapeDtypeStruct((1, 1), jnp.float32),
        scratch_shapes=[pltpu.SMEM((1,), jnp.int32)],
        compiler_params=pltpu.CompilerParams(
            dimension_semantics=("arbitrary",)),
    )(y_pred, tvals, ytrue_f)


def kernel(y_pred, y_true):
    yt = y_true.astype(jnp.int32)
    tvals = _gather_targets(y_pred.reshape(-1), yt)
    out = _count_ranks(
        y_pred,
        jnp.broadcast_to(tvals.reshape(B, 1), (B, 128)),
        jnp.broadcast_to(yt.astype(jnp.float32).reshape(B, 1), (B, 128)))
    return out[0, 0]


# speedup vs baseline: 2.0218x; 2.0218x over previous
import functools

import jax
import jax.numpy as jnp
from jax import lax
from jax.experimental import pallas as pl
from jax.experimental.pallas import tpu as pltpu
from jax.experimental.pallas import tpu_sc as plsc

B = 128
V = 100000
TOP_K = 8
RB = 8
GSTEPS = B // RB


ROWS_PER_W = 4


def _gather_kernel(pred_hbm, ytrue_hbm, out_hbm, yt_v, bwin_v, tl_v, semb):
    wid = lax.axis_index("s") * 2 + lax.axis_index("c")
    lanes = lax.iota(jnp.int32, 16)
    r0 = wid * ROWS_PER_W

    pltpu.sync_copy(ytrue_hbm, yt_v)
    ytw = yt_v[pl.ds((r0 // 16) * 16, 16)]
    win_copies = []
    starts = []
    for i in range(ROWS_PER_W):
        lane_i = r0 - (r0 // 16) * 16 + i
        yt = jnp.max(jnp.where(lanes == lane_i, ytw, 0))
        start = jnp.minimum((yt // 8) * 8, V - 16)
        win_copies.append(pltpu.async_copy(
            pred_hbm.at[r0 + i, pl.ds(start, 16)], bwin_v.at[i], semb))
        starts.append((yt, start))
    tv = jnp.zeros((16,), jnp.float32)
    for i in range(ROWS_PER_W):
        win_copies[i].wait()
        bv = bwin_v[i]
        yt, start = starts[i]
        t = jnp.max(jnp.where(lanes == yt - start, bv, jnp.float32(-3e38)))
        tv = jnp.where(lanes == i, jnp.full((16,), t, jnp.float32), tv)
    tl_v[...] = tv
    pltpu.sync_copy(tl_v, out_hbm.at[wid])


@jax.jit
def _gather_targets(pred2d, ytrue):
    mesh = plsc.VectorSubcoreMesh(core_axis_name="c", subcore_axis_name="s")
    kern = functools.partial(
        pl.kernel,
        mesh=mesh,
        compiler_params=pltpu.CompilerParams(needs_layout_passes=False),
        out_type=jax.ShapeDtypeStruct((B // ROWS_PER_W, 16), jnp.float32),
        scratch_types=[
            pltpu.VMEM((B,), jnp.int32),
            pltpu.VMEM((ROWS_PER_W, 16), jnp.float32),
            pltpu.VMEM((16,), jnp.float32),
            pltpu.SemaphoreType.DMA,
        ],
    )(_gather_kernel)
    return kern(pred2d, ytrue)


def _count_kernel(pred_ref, t_ref, yt_ref, out_ref, s_ref):
    g = pl.program_id(0)

    @pl.when(g == 0)
    def _():
        s_ref[0] = 0

    v = pred_ref[...]
    col = lax.broadcasted_iota(jnp.int32, (RB, V), 1).astype(jnp.float32)
    yt = yt_ref[:, 0:1]
    t = jnp.sum(jnp.where(col == yt, v, jnp.float32(0)), axis=1,
                keepdims=True)
    m = (v > t) | ((v == t) & (col < yt))
    rank = jnp.sum(m.astype(jnp.int32), axis=1, keepdims=True)
    s_ref[0] += jnp.sum((rank < TOP_K).astype(jnp.int32))

    @pl.when(g == GSTEPS - 1)
    def _():
        out_ref[...] = jnp.full(
            (1, 1), s_ref[0].astype(jnp.float32) / jnp.float32(B))


@jax.jit
def _count_ranks(y_pred, tvals, ytrue_f):
    return pl.pallas_call(
        _count_kernel,
        grid=(GSTEPS,),
        in_specs=[
            pl.BlockSpec((RB, V), lambda g: (g, 0)),
            pl.BlockSpec((RB, 128), lambda g: (g, 0)),
            pl.BlockSpec((RB, 128), lambda g: (g, 0)),
        ],
        out_specs=pl.BlockSpec((1, 1), lambda g: (0, 0)),
        out_shape=jax.ShapeDtypeStruct((1, 1), jnp.float32),
        scratch_shapes=[pltpu.SMEM((1,), jnp.int32)],
        compiler_params=pltpu.CompilerParams(
            dimension_semantics=("arbitrary",)),
    )(y_pred, tvals, ytrue_f)


def kernel(y_pred, y_true):
    yt = y_true.astype(jnp.int32)
    ytf = jnp.broadcast_to(yt.astype(jnp.float32).reshape(B, 1), (B, 128))
    out = _count_ranks(y_pred, ytf, ytf)
    return out[0, 0]


# speedup vs baseline: 3.2326x; 1.5989x over previous
import functools

import jax
import jax.numpy as jnp
from jax import lax
from jax.experimental import pallas as pl
from jax.experimental.pallas import tpu as pltpu
from jax.experimental.pallas import tpu_sc as plsc

B = 128
V = 100000
TOP_K = 8
ROWS_PER_W = 4
CR = 5000
GSTEPS = V // CR


def _gather_kernel(predt_hbm, ytrue_hbm, out_hbm, yt_v, win_v, tl_v, semb):
    wid = lax.axis_index("s") * 2 + lax.axis_index("c")
    lanes = lax.iota(jnp.int32, 16)
    r0 = wid * ROWS_PER_W

    pltpu.sync_copy(ytrue_hbm, yt_v)
    ytw = yt_v[pl.ds((r0 // 16) * 16, 16)]
    win_copies = []
    yts = []
    for i in range(ROWS_PER_W):
        lane_i = r0 - (r0 // 16) * 16 + i
        yt = jnp.max(jnp.where(lanes == lane_i, ytw, 0))
        rowblk = pl.multiple_of((yt // 8) * 8, 8)
        win_copies.append(pltpu.async_copy(
            predt_hbm.at[pl.ds(rowblk, 8), :], win_v.at[i], semb))
        yts.append(yt)
    tv = jnp.zeros((16,), jnp.float32)
    for i in range(ROWS_PER_W):
        win_copies[i].wait()
        b = r0 + i
        sub = yts[i] - (yts[i] // 8) * 8
        t = jnp.float32(-3e38)
        for s in range(8):
            piece = win_v[i, s, pl.ds((b // 16) * 16, 16)]
            val_s = jnp.max(jnp.where(lanes == b % 16, piece,
                                      jnp.float32(-3e38)))
            t = jnp.where(sub == s, val_s, t)
        tv = jnp.where(lanes == i, jnp.full((16,), t, jnp.float32), tv)
    tl_v[...] = tv
    pltpu.sync_copy(tl_v, out_hbm.at[wid])


@jax.jit
def _gather_targets(predt, ytrue):
    mesh = plsc.VectorSubcoreMesh(core_axis_name="c", subcore_axis_name="s")
    kern = functools.partial(
        pl.kernel,
        mesh=mesh,
        compiler_params=pltpu.CompilerParams(needs_layout_passes=False),
        out_type=jax.ShapeDtypeStruct((B // ROWS_PER_W, 16), jnp.float32),
        scratch_types=[
            pltpu.VMEM((B,), jnp.int32),
            pltpu.VMEM((ROWS_PER_W, 8, 128), jnp.float32),
            pltpu.VMEM((16,), jnp.float32),
            pltpu.SemaphoreType.DMA,
        ],
    )(_gather_kernel)
    return kern(predt, ytrue)


def _count_kernel(predt_ref, t_ref, yt_ref, out_ref, cnt_ref):
    g = pl.program_id(0)

    @pl.when(g == 0)
    def _():
        cnt_ref[...] = jnp.zeros_like(cnt_ref)

    v = predt_ref[...]
    row = (lax.broadcasted_iota(jnp.int32, (CR, 128), 0) + g * CR
           ).astype(jnp.float32)
    t = t_ref[0:1, :]
    yt = yt_ref[0:1, :]
    m = (v > t) | ((v == t) & (row < yt))
    cnt_ref[0:1, :] += jnp.sum(m.astype(jnp.int32), axis=0, keepdims=True)

    @pl.when(g == GSTEPS - 1)
    def _():
        hits = (cnt_ref[0:1, :] < TOP_K).astype(jnp.float32)
        out_ref[...] = jnp.full((1, 1), jnp.sum(hits) / jnp.float32(B))


@jax.jit
def _count_ranks(predt, tvals, ytrue_f):
    return pl.pallas_call(
        _count_kernel,
        grid=(GSTEPS,),
        in_specs=[
            pl.BlockSpec((CR, 128), lambda g: (g, 0)),
            pl.BlockSpec((8, 128), lambda g: (0, 0)),
            pl.BlockSpec((8, 128), lambda g: (0, 0)),
        ],
        out_specs=pl.BlockSpec((1, 1), lambda g: (0, 0)),
        out_shape=jax.ShapeDtypeStruct((1, 1), jnp.float32),
        scratch_shapes=[pltpu.VMEM((8, 128), jnp.int32)],
        compiler_params=pltpu.CompilerParams(
            dimension_semantics=("arbitrary",)),
    )(predt, tvals, ytrue_f)


def kernel(y_pred, y_true):
    yt = y_true.astype(jnp.int32)
    predt = y_pred.T
    tgrid = _gather_targets(predt, yt)
    tvals = tgrid[:, :ROWS_PER_W].reshape(1, B)
    out = _count_ranks(
        predt,
        jnp.broadcast_to(tvals, (8, B)),
        jnp.broadcast_to(yt.astype(jnp.float32).reshape(1, B), (8, B)))
    return out[0, 0]


# speedup vs baseline: 3.5794x; 1.1073x over previous
import functools

import jax
import jax.numpy as jnp
from jax import lax
from jax.experimental import pallas as pl
from jax.experimental.pallas import tpu as pltpu
from jax.experimental.pallas import tpu_sc as plsc

B = 128
V = 100000
TOP_K = 8
ROWS_PER_W = 4
CR = 10000
GSTEPS = V // CR


def _gather_kernel(predt_hbm, ytrue_hbm, out_hbm, yt_v, win_v, tl_v, semb):
    wid = lax.axis_index("s") * 2 + lax.axis_index("c")
    lanes = lax.iota(jnp.int32, 16)
    r0 = wid * ROWS_PER_W

    pltpu.sync_copy(ytrue_hbm, yt_v)
    ytw = yt_v[pl.ds((r0 // 16) * 16, 16)]
    win_copies = []
    yts = []
    for i in range(ROWS_PER_W):
        lane_i = r0 - (r0 // 16) * 16 + i
        yt = jnp.max(jnp.where(lanes == lane_i, ytw, 0))
        rowblk = pl.multiple_of((yt // 8) * 8, 8)
        win_copies.append(pltpu.async_copy(
            predt_hbm.at[pl.ds(rowblk, 8), :], win_v.at[i], semb))
        yts.append(yt)
    tv = jnp.zeros((16,), jnp.float32)
    for i in range(ROWS_PER_W):
        win_copies[i].wait()
        b = r0 + i
        sub = yts[i] - (yts[i] // 8) * 8
        t = jnp.float32(-3e38)
        for s in range(8):
            piece = win_v[i, s, pl.ds((b // 16) * 16, 16)]
            val_s = jnp.max(jnp.where(lanes == b % 16, piece,
                                      jnp.float32(-3e38)))
            t = jnp.where(sub == s, val_s, t)
        tv = jnp.where(lanes == i, jnp.full((16,), t, jnp.float32), tv)
    tl_v[...] = tv
    pltpu.sync_copy(tl_v, out_hbm.at[wid])


@jax.jit
def _gather_targets(predt, ytrue):
    mesh = plsc.VectorSubcoreMesh(core_axis_name="c", subcore_axis_name="s")
    kern = functools.partial(
        pl.kernel,
        mesh=mesh,
        compiler_params=pltpu.CompilerParams(needs_layout_passes=False),
        out_type=jax.ShapeDtypeStruct((B // ROWS_PER_W, 16), jnp.float32),
        scratch_types=[
            pltpu.VMEM((B,), jnp.int32),
            pltpu.VMEM((ROWS_PER_W, 8, 128), jnp.float32),
            pltpu.VMEM((16,), jnp.float32),
            pltpu.SemaphoreType.DMA,
        ],
    )(_gather_kernel)
    return kern(predt, ytrue)


def _count_kernel(predt_ref, t_ref, yt_ref, out_ref, cnt_ref):
    g = pl.program_id(0)

    @pl.when(g == 0)
    def _():
        cnt_ref[...] = jnp.zeros_like(cnt_ref)

    v = predt_ref[...]
    row = (lax.broadcasted_iota(jnp.int32, (CR, 128), 0) + g * CR
           ).astype(jnp.float32)
    t = t_ref[0:1, :]
    yt = yt_ref[0:1, :]
    m = (v > t) | ((v == t) & (row < yt))
    cnt_ref[0:1, :] += jnp.sum(m.astype(jnp.int32), axis=0, keepdims=True)

    @pl.when(g == GSTEPS - 1)
    def _():
        hits = (cnt_ref[0:1, :] < TOP_K).astype(jnp.float32)
        out_ref[...] = jnp.full((1, 1), jnp.sum(hits) / jnp.float32(B))


@jax.jit
def _count_ranks(predt, tvals, ytrue_f):
    return pl.pallas_call(
        _count_kernel,
        grid=(GSTEPS,),
        in_specs=[
            pl.BlockSpec((CR, 128), lambda g: (g, 0)),
            pl.BlockSpec((8, 128), lambda g: (0, 0)),
            pl.BlockSpec((8, 128), lambda g: (0, 0)),
        ],
        out_specs=pl.BlockSpec((1, 1), lambda g: (0, 0)),
        out_shape=jax.ShapeDtypeStruct((1, 1), jnp.float32),
        scratch_shapes=[pltpu.VMEM((8, 128), jnp.int32)],
        compiler_params=pltpu.CompilerParams(
            dimension_semantics=("arbitrary",)),
    )(predt, tvals, ytrue_f)


def kernel(y_pred, y_true):
    yt = y_true.astype(jnp.int32)
    predt = y_pred.T
    tgrid = _gather_targets(predt, yt)
    tvals = tgrid[:, :ROWS_PER_W].reshape(1, B)
    out = _count_ranks(
        predt,
        jnp.broadcast_to(tvals, (8, B)),
        jnp.broadcast_to(yt.astype(jnp.float32).reshape(1, B), (8, B)))
    return out[0, 0]


# speedup vs baseline: 3.6391x; 1.0167x over previous
import functools

import jax
import jax.numpy as jnp
from jax import lax
from jax.experimental import pallas as pl
from jax.experimental.pallas import tpu as pltpu
from jax.experimental.pallas import tpu_sc as plsc

B = 128
V = 100000
TOP_K = 8
ROWS_PER_W = 4
CR = 20000
GSTEPS = V // CR


def _gather_kernel(predt_hbm, ytrue_hbm, out_hbm, yt_v, win_v, tl_v, semb):
    wid = lax.axis_index("s") * 2 + lax.axis_index("c")
    lanes = lax.iota(jnp.int32, 16)
    r0 = wid * ROWS_PER_W

    pltpu.sync_copy(ytrue_hbm, yt_v)
    ytw = yt_v[pl.ds((r0 // 16) * 16, 16)]
    win_copies = []
    yts = []
    for i in range(ROWS_PER_W):
        lane_i = r0 - (r0 // 16) * 16 + i
        yt = jnp.max(jnp.where(lanes == lane_i, ytw, 0))
        rowblk = pl.multiple_of((yt // 8) * 8, 8)
        win_copies.append(pltpu.async_copy(
            predt_hbm.at[pl.ds(rowblk, 8), :], win_v.at[i], semb))
        yts.append(yt)
    tv = jnp.zeros((16,), jnp.float32)
    for i in range(ROWS_PER_W):
        win_copies[i].wait()
        b = r0 + i
        sub = yts[i] - (yts[i] // 8) * 8
        t = jnp.float32(-3e38)
        for s in range(8):
            piece = win_v[i, s, pl.ds((b // 16) * 16, 16)]
            val_s = jnp.max(jnp.where(lanes == b % 16, piece,
                                      jnp.float32(-3e38)))
            t = jnp.where(sub == s, val_s, t)
        tv = jnp.where(lanes == i, jnp.full((16,), t, jnp.float32), tv)
    tl_v[...] = tv
    pltpu.sync_copy(tl_v, out_hbm.at[wid])


@jax.jit
def _gather_targets(predt, ytrue):
    mesh = plsc.VectorSubcoreMesh(core_axis_name="c", subcore_axis_name="s")
    kern = functools.partial(
        pl.kernel,
        mesh=mesh,
        compiler_params=pltpu.CompilerParams(needs_layout_passes=False),
        out_type=jax.ShapeDtypeStruct((B // ROWS_PER_W, 16), jnp.float32),
        scratch_types=[
            pltpu.VMEM((B,), jnp.int32),
            pltpu.VMEM((ROWS_PER_W, 8, 128), jnp.float32),
            pltpu.VMEM((16,), jnp.float32),
            pltpu.SemaphoreType.DMA,
        ],
    )(_gather_kernel)
    return kern(predt, ytrue)


def _count_kernel(predt_ref, t_ref, yt_ref, out_ref, cnt_ref):
    g = pl.program_id(0)

    @pl.when(g == 0)
    def _():
        cnt_ref[...] = jnp.zeros_like(cnt_ref)

    v = predt_ref[...]
    row = (lax.broadcasted_iota(jnp.int32, (CR, 128), 0) + g * CR
           ).astype(jnp.float32)
    t = t_ref[0:1, :]
    yt = yt_ref[0:1, :]
    m = (v > t) | ((v == t) & (row < yt))
    cnt_ref[0:1, :] += jnp.sum(m.astype(jnp.int32), axis=0, keepdims=True)

    @pl.when(g == GSTEPS - 1)
    def _():
        hits = (cnt_ref[0:1, :] < TOP_K).astype(jnp.float32)
        out_ref[...] = jnp.full((1, 1), jnp.sum(hits) / jnp.float32(B))


@jax.jit
def _count_ranks(predt, tvals, ytrue_f):
    return pl.pallas_call(
        _count_kernel,
        grid=(GSTEPS,),
        in_specs=[
            pl.BlockSpec((CR, 128), lambda g: (g, 0)),
            pl.BlockSpec((8, 128), lambda g: (0, 0)),
            pl.BlockSpec((8, 128), lambda g: (0, 0)),
        ],
        out_specs=pl.BlockSpec((1, 1), lambda g: (0, 0)),
        out_shape=jax.ShapeDtypeStruct((1, 1), jnp.float32),
        scratch_shapes=[pltpu.VMEM((8, 128), jnp.int32)],
        compiler_params=pltpu.CompilerParams(
            dimension_semantics=("arbitrary",)),
    )(predt, tvals, ytrue_f)


def kernel(y_pred, y_true):
    yt = y_true.astype(jnp.int32)
    predt = y_pred.T
    tgrid = _gather_targets(predt, yt)
    tvals = tgrid[:, :ROWS_PER_W].reshape(1, B)
    out = _count_ranks(
        predt,
        jnp.broadcast_to(tvals, (8, B)),
        jnp.broadcast_to(yt.astype(jnp.float32).reshape(1, B), (8, B)))
    return out[0, 0]
